# baseline (speedup 1.0000x reference)
import functools

import jax
import jax.numpy as jnp
from jax import lax
from jax.experimental import pallas as pl
from jax.experimental.pallas import tpu as pltpu
from jax.experimental.pallas import tpu_sc as plsc

N = 5000
NPAD = 5120
D_IN = 128
E = 320000
CH = 128
NCH = 80
UNR = 4
NSUB = 16
NCORE = 2
EPAD = NCORE * NSUB * NCH * CH
RPT = NPAD // NSUB

_f32 = jnp.float32
_MESH = dict(core_axis_name="c", subcore_axis_name="s")


def _sc_degrees():
  @functools.partial(
      pl.kernel,
      mesh=plsc.VectorSubcoreMesh(**_MESH),
      compiler_params=pltpu.CompilerParams(use_tc_tiling_on_sc=False),
      out_type=jax.ShapeDtypeStruct((4, NCORE, NPAD, 16), _f32),
      scratch_types=[
          pltpu.VMEM_SHARED((NPAD, 16), _f32),
          pltpu.VMEM_SHARED((NPAD, 16), _f32),
          pltpu.VMEM_SHARED((NPAD, 16), _f32),
          pltpu.VMEM_SHARED((NPAD, 16), _f32),
          pltpu.VMEM((4, NCH, CH), jnp.int32),
          pltpu.VMEM((CH, 16), _f32),
          pltpu.SemaphoreType.DMA,
      ],
  )
  def deg_kernel(idx4, ones_h, zeros_h, degp, a0, a1, a2, a3, idx_v, ones_v,
                 sem):
    cid = lax.axis_index("c")
    sid = lax.axis_index("s")
    wid = cid * NSUB + sid
    row0 = sid * RPT
    pltpu.sync_copy(ones_h, ones_v)
    for a in range(4):
      pltpu.sync_copy(idx4.at[a, wid], idx_v.at[a])
    accs = (a0, a1, a2, a3)
    for acc in accs:
      pltpu.sync_copy(zeros_h.at[pl.ds(row0, RPT)], acc.at[pl.ds(row0, RPT)])
    plsc.subcore_barrier()
    for a, acc in enumerate(accs):
      def body(j, carry, a=a, acc=acc):
        pltpu.async_copy(ones_v, acc.at[idx_v.at[a, j]], sem, add=True)
        return carry

      lax.fori_loop(0, NCH, body, 0)

    def drain(j, carry):
      pltpu.make_async_copy(ones_h, ones_v, sem).wait()
      return carry

    lax.fori_loop(0, 4 * NCH, drain, 0)
    plsc.subcore_barrier()
    for a, acc in enumerate(accs):
      pltpu.sync_copy(acc.at[pl.ds(row0, RPT)],
                      degp.at[a, cid, pl.ds(row0, RPT)])

  return deg_kernel


def _sc_conv(D):
  @functools.partial(
      pl.kernel,
      mesh=plsc.VectorSubcoreMesh(**_MESH),
      compiler_params=pltpu.CompilerParams(use_tc_tiling_on_sc=False),
      out_type=[
          jax.ShapeDtypeStruct((NCORE, NPAD, D), _f32),
          jax.ShapeDtypeStruct((NCORE, NPAD, D), _f32),
      ],
      scratch_types=[
          pltpu.VMEM_SHARED((NPAD, D), _f32),
          pltpu.VMEM_SHARED((NPAD, D), _f32),
          pltpu.VMEM((NCH, CH), jnp.int32),
          pltpu.VMEM((NCH, CH), jnp.int32),
      ] + [pltpu.VMEM((CH, D), _f32)] * UNR
        + [pltpu.SemaphoreType.DMA] * (2 * UNR),
  )
  def conv_kernel(m_dd, m_rev, idx_dd, idx_rev, zeros_h, agg_dd, agg_rev,
                  acc, tbl, src_v, dst_v, *bufsem):
    bufs = bufsem[:UNR]
    gsem = bufsem[UNR:2 * UNR]
    ssem = bufsem[2 * UNR:]
    cid = lax.axis_index("c")
    sid = lax.axis_index("s")
    wid = cid * NSUB + sid
    row0 = sid * RPT
    for m, idx, agg in ((m_dd, idx_dd, agg_dd), (m_rev, idx_rev, agg_rev)):
      pltpu.sync_copy(zeros_h.at[pl.ds(row0, RPT)], acc.at[pl.ds(row0, RPT)])
      pltpu.sync_copy(m.at[pl.ds(row0, RPT)], tbl.at[pl.ds(row0, RPT)])
      pltpu.sync_copy(idx.at[0, wid], src_v)
      pltpu.sync_copy(idx.at[1, wid], dst_v)
      plsc.subcore_barrier()

      def body(r, carry, m=m):
        gh = []
        for b in range(UNR):
          @pl.when(r > 0)
          def _(b=b):
            pltpu.make_async_copy(m.at[pl.ds(0, CH)], bufs[b], ssem[b]).wait()
          gh.append(pltpu.async_copy(
              tbl.at[src_v.at[r * UNR + b]], bufs[b], gsem[b]))
        for b in range(UNR):
          gh[b].wait()
          pltpu.async_copy(
              bufs[b], acc.at[dst_v.at[r * UNR + b]], ssem[b], add=True)
        return carry

      lax.fori_loop(0, NCH // UNR, body, 0)
      for b in range(UNR):
        pltpu.make_async_copy(m.at[pl.ds(0, CH)], bufs[b], ssem[b]).wait()
      plsc.subcore_barrier()
      pltpu.sync_copy(acc.at[pl.ds(row0, RPT)], agg.at[cid, pl.ds(row0, RPT)])

  return conv_kernel


def _tc1_body(fd, fs, wde, wdi, w1d, w1r, dp,
              m1d_o, m1r_o, rsod_o, rsid_o, rsor_o, rsir_o):
  def rs(a):
    return lax.rsqrt(jnp.maximum(dp[a, 0] + dp[a, 1], 1.0))

  rsod, rsid, rsor, rsir = rs(0), rs(1), rs(2), rs(3)
  hd = jnp.dot(fd[...], wde[...], preferred_element_type=_f32)
  hs = jnp.dot(fs[...], wdi[...], preferred_element_type=_f32)
  m1d_o[...] = rsod[:, 0:1] * jnp.dot(hd, w1d[...], preferred_element_type=_f32)
  m1r_o[...] = rsor[:, 0:1] * jnp.dot(hs, w1r[...], preferred_element_type=_f32)
  rsod_o[...] = rsod
  rsid_o[...] = rsid
  rsor_o[...] = rsor
  rsir_o[...] = rsir


def _tc2_body(a1d, a1r, rsid, rsir, rsod, rsor, b1d, b1r, w2d, w2r,
              m2d_o, m2r_o):
  h_dis1 = jnp.maximum((a1d[0] + a1d[1]) * rsid[:, 0:1] + b1d[...], 0.0)
  h_drug1 = jnp.maximum((a1r[0] + a1r[1]) * rsir[:, 0:1] + b1r[...], 0.0)
  m2d_o[...] = rsod[:, 0:1] * jnp.dot(h_drug1, w2d[...],
                                      preferred_element_type=_f32)
  m2r_o[...] = rsor[:, 0:1] * jnp.dot(h_dis1, w2r[...],
                                      preferred_element_type=_f32)


MB = 1024


def _tc3_body(a2d, a2r, rsid, rsir, b2d, b2r, wout, out_o):
  h_dis2 = jnp.maximum((a2d[0] + a2d[1]) * rsid[:, 0:1] + b2d[...], 0.0)
  h_drug2 = jnp.maximum((a2r[0] + a2r[1]) * rsir[:, 0:1] + b2r[...], 0.0)
  a = jnp.dot(h_drug2, wout[...], preferred_element_type=_f32)
  out_o[...] = lax.dot_general(a, h_dis2[0:N], (((1,), (1,)), ((), ())),
                               preferred_element_type=_f32)


def _pad_edges(edge_index):
  pad = N + (jnp.arange(EPAD - E, dtype=jnp.int32) % (NPAD - N))
  src = jnp.concatenate([edge_index[0].astype(jnp.int32), pad])
  dst = jnp.concatenate([edge_index[1].astype(jnp.int32), pad])
  return jnp.stack([src, dst]).reshape(2, NCORE * NSUB, NCH, CH)


def kernel(feature_drug, feature_disease, edge_index_drug_disease,
           edge_index_disease_drug, W_drug_emb, W_dis_emb, W1_dd, b1_dd,
           W1_rev, b1_rev, W2_dd, b2_dd, W2_rev, b2_rev, W_out):
  idx_dd = _pad_edges(edge_index_drug_disease)
  idx_rev = _pad_edges(edge_index_disease_drug)
  idx4 = jnp.concatenate([idx_dd, idx_rev], axis=0)
  fd = jnp.pad(feature_drug, ((0, NPAD - N), (0, 0)))
  fs = jnp.pad(feature_disease, ((0, NPAD - N), (0, 0)))
  ones16 = jnp.ones((CH, 16), _f32)
  zeros16 = jnp.zeros((NPAD, 16), _f32)
  zeros64 = jnp.zeros((NPAD, 64), _f32)
  zeros32 = jnp.zeros((NPAD, 32), _f32)

  degp = _sc_degrees()(idx4, ones16, zeros16)

  shapes1 = ([jax.ShapeDtypeStruct((NPAD, 64), _f32)] * 2 +
             [jax.ShapeDtypeStruct((NPAD, 16), _f32)] * 4)
  m1d, m1r, rsod, rsid, rsor, rsir = pl.pallas_call(
      _tc1_body, out_shape=shapes1)(
          fd, fs, W_drug_emb, W_dis_emb, W1_dd, W1_rev, degp)

  a1d, a1r = _sc_conv(64)(m1d, m1r, idx_dd, idx_rev, zeros64)

  m2d, m2r = pl.pallas_call(
      _tc2_body, out_shape=[jax.ShapeDtypeStruct((NPAD, 32), _f32)] * 2)(
          a1d, a1r, rsid, rsir, rsod, rsor,
          b1_dd.reshape(1, 64), b1_rev.reshape(1, 64), W2_dd, W2_rev)

  a2d, a2r = _sc_conv(32)(m2d, m2r, idx_dd, idx_rev, zeros32)

  out = pl.pallas_call(
      _tc3_body,
      grid=(NPAD // MB,),
      in_specs=[
          pl.BlockSpec((NCORE, NPAD, 32), lambda i: (0, 0, 0)),
          pl.BlockSpec((NCORE, MB, 32), lambda i: (0, i, 0)),
          pl.BlockSpec((NPAD, 16), lambda i: (0, 0)),
          pl.BlockSpec((MB, 16), lambda i: (i, 0)),
          pl.BlockSpec((1, 32), lambda i: (0, 0)),
          pl.BlockSpec((1, 32), lambda i: (0, 0)),
          pl.BlockSpec((32, 32), lambda i: (0, 0)),
      ],
      out_specs=pl.BlockSpec((MB, N), lambda i: (i, 0)),
      out_shape=jax.ShapeDtypeStruct((N, N), _f32),
  )(a2d, a2r, rsid, rsir, b2_dd.reshape(1, 32), b2_rev.reshape(1, 32), W_out)
  return out


# speedup vs baseline: 1.1699x; 1.1699x over previous
import functools

import jax
import jax.numpy as jnp
from jax import lax
from jax.experimental import pallas as pl
from jax.experimental.pallas import tpu as pltpu
from jax.experimental.pallas import tpu_sc as plsc

N = 5000
NPAD = 5120
D_IN = 128
E = 320000
CH = 128
NCH = 80
UNR = 4
NSUB = 16
NCORE = 2
EPAD = NCORE * NSUB * NCH * CH
RPT = NPAD // NSUB

_f32 = jnp.float32
_MESH = dict(core_axis_name="c", subcore_axis_name="s")


def _sc_degrees():
  @functools.partial(
      pl.kernel,
      mesh=plsc.VectorSubcoreMesh(**_MESH),
      compiler_params=pltpu.CompilerParams(use_tc_tiling_on_sc=False),
      out_type=jax.ShapeDtypeStruct((4, NCORE, NPAD, 16), _f32),
      scratch_types=[
          pltpu.VMEM_SHARED((NPAD, 16), _f32),
          pltpu.VMEM_SHARED((NPAD, 16), _f32),
          pltpu.VMEM_SHARED((NPAD, 16), _f32),
          pltpu.VMEM_SHARED((NPAD, 16), _f32),
          pltpu.VMEM((4, NCH, CH), jnp.int32),
          pltpu.VMEM((CH, 16), _f32),
          pltpu.SemaphoreType.DMA,
      ],
  )
  def deg_kernel(idx4, ones_h, zeros_h, degp, a0, a1, a2, a3, idx_v, ones_v,
                 sem):
    cid = lax.axis_index("c")
    sid = lax.axis_index("s")
    wid = cid * NSUB + sid
    row0 = sid * RPT
    pltpu.sync_copy(ones_h, ones_v)
    for a in range(4):
      pltpu.sync_copy(idx4.at[a, wid], idx_v.at[a])
    accs = (a0, a1, a2, a3)
    for acc in accs:
      pltpu.sync_copy(zeros_h.at[pl.ds(row0, RPT)], acc.at[pl.ds(row0, RPT)])
    plsc.subcore_barrier()
    for a, acc in enumerate(accs):
      def body(j, carry, a=a, acc=acc):
        pltpu.async_copy(ones_v, acc.at[idx_v.at[a, j]], sem, add=True)
        return carry

      lax.fori_loop(0, NCH, body, 0)

    def drain(j, carry):
      pltpu.make_async_copy(ones_h, ones_v, sem).wait()
      return carry

    lax.fori_loop(0, 4 * NCH, drain, 0)
    plsc.subcore_barrier()
    for a, acc in enumerate(accs):
      pltpu.sync_copy(acc.at[pl.ds(row0, RPT)],
                      degp.at[a, cid, pl.ds(row0, RPT)])

  return deg_kernel


def _sc_conv(D):
  @functools.partial(
      pl.kernel,
      mesh=plsc.VectorSubcoreMesh(**_MESH),
      compiler_params=pltpu.CompilerParams(use_tc_tiling_on_sc=False),
      out_type=[
          jax.ShapeDtypeStruct((NCORE, NPAD, D), _f32),
          jax.ShapeDtypeStruct((NCORE, NPAD, D), _f32),
      ],
      scratch_types=[
          pltpu.VMEM_SHARED((NPAD, D), _f32),
          pltpu.VMEM_SHARED((NPAD, D), _f32),
          pltpu.VMEM((NCH, CH), jnp.int32),
          pltpu.VMEM((NCH, CH), jnp.int32),
      ] + [pltpu.VMEM((CH, D), _f32)] * UNR
        + [pltpu.SemaphoreType.DMA] * (2 * UNR),
  )
  def conv_kernel(m_dd, m_rev, idx_dd, idx_rev, zeros_h, agg_dd, agg_rev,
                  acc, tbl, src_v, dst_v, *bufsem):
    bufs = bufsem[:UNR]
    gsem = bufsem[UNR:2 * UNR]
    ssem = bufsem[2 * UNR:]
    cid = lax.axis_index("c")
    sid = lax.axis_index("s")
    wid = cid * NSUB + sid
    row0 = sid * RPT
    for m, idx, agg in ((m_dd, idx_dd, agg_dd), (m_rev, idx_rev, agg_rev)):
      pltpu.sync_copy(zeros_h.at[pl.ds(row0, RPT)], acc.at[pl.ds(row0, RPT)])
      pltpu.sync_copy(m.at[pl.ds(row0, RPT)], tbl.at[pl.ds(row0, RPT)])
      pltpu.sync_copy(idx.at[0, wid], src_v)
      pltpu.sync_copy(idx.at[1, wid], dst_v)
      plsc.subcore_barrier()

      def body(r, carry):
        gh = []
        for b in range(UNR):
          gh.append(pltpu.async_copy(
              tbl.at[src_v.at[r * UNR + b]], bufs[b], gsem[b]))
        sh = []
        for b in range(UNR):
          gh[b].wait()
          sh.append(pltpu.async_copy(
              bufs[b], acc.at[dst_v.at[r * UNR + b]], ssem[b], add=True))
        for b in range(UNR):
          sh[b].wait()
        return carry

      lax.fori_loop(0, NCH // UNR, body, 0)
      plsc.subcore_barrier()
      pltpu.sync_copy(acc.at[pl.ds(row0, RPT)], agg.at[cid, pl.ds(row0, RPT)])

  return conv_kernel


def _tc1_body(fd, fs, wde, wdi, w1d, w1r, dp,
              m1d_o, m1r_o, rsod_o, rsid_o, rsor_o, rsir_o):
  def rs(a):
    return lax.rsqrt(jnp.maximum(dp[a, 0] + dp[a, 1], 1.0))

  rsod, rsid, rsor, rsir = rs(0), rs(1), rs(2), rs(3)
  hd = jnp.dot(fd[...], wde[...], preferred_element_type=_f32)
  hs = jnp.dot(fs[...], wdi[...], preferred_element_type=_f32)
  m1d_o[...] = rsod[:, 0:1] * jnp.dot(hd, w1d[...], preferred_element_type=_f32)
  m1r_o[...] = rsor[:, 0:1] * jnp.dot(hs, w1r[...], preferred_element_type=_f32)
  rsod_o[...] = rsod
  rsid_o[...] = rsid
  rsor_o[...] = rsor
  rsir_o[...] = rsir


def _tc2_body(a1d, a1r, rsid, rsir, rsod, rsor, b1d, b1r, w2d, w2r,
              m2d_o, m2r_o):
  h_dis1 = jnp.maximum((a1d[0] + a1d[1]) * rsid[:, 0:1] + b1d[...], 0.0)
  h_drug1 = jnp.maximum((a1r[0] + a1r[1]) * rsir[:, 0:1] + b1r[...], 0.0)
  m2d_o[...] = rsod[:, 0:1] * jnp.dot(h_drug1, w2d[...],
                                      preferred_element_type=_f32)
  m2r_o[...] = rsor[:, 0:1] * jnp.dot(h_dis1, w2r[...],
                                      preferred_element_type=_f32)


MB = 1024


def _tc3_body(a2d, a2r, rsid, rsir, b2d, b2r, wout, out_o):
  h_dis2 = jnp.maximum((a2d[0] + a2d[1]) * rsid[:, 0:1] + b2d[...], 0.0)
  h_drug2 = jnp.maximum((a2r[0] + a2r[1]) * rsir[:, 0:1] + b2r[...], 0.0)
  a = jnp.dot(h_drug2, wout[...], preferred_element_type=_f32)
  out_o[...] = lax.dot_general(a, h_dis2[0:N], (((1,), (1,)), ((), ())),
                               preferred_element_type=_f32)


def _pad_edges(edge_index):
  pad = N + (jnp.arange(EPAD - E, dtype=jnp.int32) % (NPAD - N))
  src = jnp.concatenate([edge_index[0].astype(jnp.int32), pad])
  dst = jnp.concatenate([edge_index[1].astype(jnp.int32), pad])
  return jnp.stack([src, dst]).reshape(2, NCORE * NSUB, NCH, CH)


def kernel(feature_drug, feature_disease, edge_index_drug_disease,
           edge_index_disease_drug, W_drug_emb, W_dis_emb, W1_dd, b1_dd,
           W1_rev, b1_rev, W2_dd, b2_dd, W2_rev, b2_rev, W_out):
  idx_dd = _pad_edges(edge_index_drug_disease)
  idx_rev = _pad_edges(edge_index_disease_drug)
  idx4 = jnp.concatenate([idx_dd, idx_rev], axis=0)
  fd = jnp.pad(feature_drug, ((0, NPAD - N), (0, 0)))
  fs = jnp.pad(feature_disease, ((0, NPAD - N), (0, 0)))
  ones16 = jnp.ones((CH, 16), _f32)
  zeros16 = jnp.zeros((NPAD, 16), _f32)
  zeros64 = jnp.zeros((NPAD, 64), _f32)
  zeros32 = jnp.zeros((NPAD, 32), _f32)

  degp = _sc_degrees()(idx4, ones16, zeros16)

  shapes1 = ([jax.ShapeDtypeStruct((NPAD, 64), _f32)] * 2 +
             [jax.ShapeDtypeStruct((NPAD, 16), _f32)] * 4)
  m1d, m1r, rsod, rsid, rsor, rsir = pl.pallas_call(
      _tc1_body, out_shape=shapes1)(
          fd, fs, W_drug_emb, W_dis_emb, W1_dd, W1_rev, degp)

  a1d, a1r = _sc_conv(64)(m1d, m1r, idx_dd, idx_rev, zeros64)

  m2d, m2r = pl.pallas_call(
      _tc2_body, out_shape=[jax.ShapeDtypeStruct((NPAD, 32), _f32)] * 2)(
          a1d, a1r, rsid, rsir, rsod, rsor,
          b1_dd.reshape(1, 64), b1_rev.reshape(1, 64), W2_dd, W2_rev)

  a2d, a2r = _sc_conv(32)(m2d, m2r, idx_dd, idx_rev, zeros32)

  out = pl.pallas_call(
      _tc3_body,
      grid=(NPAD // MB,),
      in_specs=[
          pl.BlockSpec((NCORE, NPAD, 32), lambda i: (0, 0, 0)),
          pl.BlockSpec((NCORE, MB, 32), lambda i: (0, i, 0)),
          pl.BlockSpec((NPAD, 16), lambda i: (0, 0)),
          pl.BlockSpec((MB, 16), lambda i: (i, 0)),
          pl.BlockSpec((1, 32), lambda i: (0, 0)),
          pl.BlockSpec((1, 32), lambda i: (0, 0)),
          pl.BlockSpec((32, 32), lambda i: (0, 0)),
      ],
      out_specs=pl.BlockSpec((MB, N), lambda i: (i, 0)),
      out_shape=jax.ShapeDtypeStruct((N, N), _f32),
  )(a2d, a2r, rsid, rsir, b2_dd.reshape(1, 32), b2_rev.reshape(1, 32), W_out)
  return out
